# baseline (speedup 1.0000x reference)
import functools
import math

import jax
import jax.numpy as jnp
from jax import lax
from jax.experimental import pallas as pl
from jax.experimental.pallas import tpu as pltpu
from jax.experimental.pallas import tpu_sc as plsc

_VOCAB = 100000
_D = 128
_BATCH = 4096
_SEQ = 50
_SEQP = 56
_B = _BATCH * _SEQ
_SCALE = math.sqrt(_D)

_K = 4
_BATCH_K = _BATCH // _K

_NC = 2
_NS = 16
_NW = _NC * _NS
_BPW = _BATCH_K // _NW
_NB = 4
_ROWS = _NB * _SEQ
_NCHUNK = _BPW // _NB

_TCG = 8
_TC_STEPS = _BATCH_K // _TCG


def _make_sc_part(part):
    @functools.partial(
        pl.kernel,
        mesh=plsc.VectorSubcoreMesh(core_axis_name="c", subcore_axis_name="s"),
        out_type=jax.ShapeDtypeStruct((_BATCH_K * _SEQ, _D), jnp.float32),
        scratch_types=(
            [pltpu.VMEM((_ROWS,), jnp.int32) for _ in range(2)]
            + [pltpu.VMEM((_ROWS, _D), jnp.float32) for _ in range(2)]
            + [pltpu.SemaphoreType.DMA for _ in range(4)]
        ),
        name=f"emb_gather{part}",
    )
    def _sc_gather(tok_hbm, table_hbm, out_hbm, idx0, idx1, rows0, rows1,
                   gsem0, gsem1, osem0, osem1):
        idx = (idx0, idx1)
        rows = (rows0, rows1)
        gsem = (gsem0, gsem1)
        osem = (osem0, osem1)

        wid = lax.axis_index("s") * _NC + lax.axis_index("c")
        base = wid * _BPW
        tok_base = part * _BATCH_K

        def start_gather(g, b):
            pltpu.sync_copy(
                tok_hbm.at[pl.ds((tok_base + base + g * _NB) * _SEQ, _ROWS)],
                idx[b])
            pltpu.async_copy(table_hbm.at[idx[b]], rows[b], gsem[b])

        def wait_gather(b):
            pltpu.make_async_copy(
                table_hbm.at[pl.ds(0, _ROWS)], rows[b], gsem[b]).wait()

        def start_out(g, b):
            pltpu.async_copy(
                rows[b],
                out_hbm.at[pl.ds((base + g * _NB) * _SEQ, _ROWS)],
                osem[b])

        def wait_out(b):
            pltpu.make_async_copy(
                rows[b], out_hbm.at[pl.ds(0, _ROWS)], osem[b]).wait()

        start_gather(0, 0)

        def pair_body(p, carry):
            g0 = p * 2
            for b in range(2):
                g = g0 + b
                nb = 1 - b
                @pl.when(g >= 1)
                def _():
                    wait_out(nb)

                @pl.when(g + 1 < _NCHUNK)
                def _():
                    start_gather(g + 1, nb)

                wait_gather(b)
                start_out(g, b)
            return carry

        lax.fori_loop(0, _NCHUNK // 2, pair_body, 0)
        wait_out(1)

    return _sc_gather


def _make_tc_part(part, aliased):
    def _tc_scale(*refs):
        part_ref, out_ref = refs[-2], refs[-1]
        x = part_ref[...].reshape(_TCG, _SEQ, _D)
        out_ref[...] = x * _SCALE

    part_spec = pl.BlockSpec((_TCG * _SEQ, _D), lambda i: (i, 0))
    in_specs = (
        [pl.BlockSpec(memory_space=pl.ANY), part_spec] if aliased
        else [part_spec])
    out_spec = pl.BlockSpec(
        (_TCG, _SEQ, _D), lambda i: (part * _TC_STEPS + i, 0, 0))
    return pl.pallas_call(
        _tc_scale,
        grid=(_TC_STEPS,),
        in_specs=in_specs,
        out_specs=out_spec,
        out_shape=jax.ShapeDtypeStruct((_BATCH, _SEQ, _D), jnp.float32),
        input_output_aliases={0: 0} if aliased else {},
        name=f"emb_scale{part}",
    )


_SC_PARTS = [_make_sc_part(k) for k in range(_K)]
_TC_PARTS = [_make_tc_part(k, aliased=(k > 0)) for k in range(_K)]


def kernel(text_tokens, table):
    flat_tok = text_tokens.reshape(_B).astype(jnp.int32)
    parts = [sc(flat_tok, table) for sc in _SC_PARTS]
    out = _TC_PARTS[0](parts[0])
    for k in range(1, _K):
        out = _TC_PARTS[k](out, parts[k])
    return out
